# baseline (speedup 1.0000x reference)
import jax
import jax.numpy as jnp
from jax import lax
from jax.experimental import pallas as pl
from jax.experimental.pallas import tpu as pltpu
from jax.experimental.pallas import tpu_sc as plsc

NUM_TOKENS = 8192
HIDDEN = 2048
NUM_EXPERTS = 16
CAPACITY = 1024
OUT_ROWS = NUM_EXPERTS * CAPACITY

NC = 2
NS = 16
NW = NC * NS
LANES = 16

TOK_PER_W = NUM_TOKENS // NW
K = 16
NDMA = TOK_PER_W // K
ZROWS = 16


def _body(th_hbm, eidx_hbm, slot_hbm, off_hbm, out_hbm,
          eidx_all, slot_v, off_v, rows_v, zeros_v,
          sem_in0, sem_in1, sem_out0, sem_out1, sem_z):
    c = lax.axis_index("c")
    s = lax.axis_index("s")
    wid = s * NC + c
    base = wid * TOK_PER_W

    iota = lax.iota(jnp.int32, LANES)
    zero16 = jnp.zeros((LANES,), jnp.float32)
    ones16 = jnp.ones((LANES,), jnp.int32)

    in_sems = (sem_in0, sem_in1)
    out_sems = (sem_out0, sem_out1)

    in_cp = {
        0: pltpu.async_copy(th_hbm.at[pl.ds(base, K)], rows_v.at[0],
                            in_sems[0]),
        1: pltpu.async_copy(th_hbm.at[pl.ds(base + K, K)], rows_v.at[1],
                            in_sems[1]),
    }

    pltpu.sync_copy(eidx_hbm, eidx_all)
    pltpu.sync_copy(slot_hbm.at[pl.ds(base, TOK_PER_W)], slot_v)
    pltpu.sync_copy(off_hbm, off_v)

    for r in range(ZROWS):
        @pl.loop(0, HIDDEN // LANES, unroll=4)
        def _zinit(i, r=r):
            zeros_v[r, pl.ds(i * LANES, LANES)] = zero16

    e_mine = wid // 2

    @pl.loop(0, NUM_TOKENS // LANES, unroll=4,
             init_carry=jnp.zeros((LANES,), jnp.int32))
    def _count(i, cnt_vec):
        e = eidx_all[pl.ds(i * LANES, LANES)]
        return cnt_vec + (e == e_mine).astype(jnp.int32)

    cnt = jnp.sum(_count)

    offs_lo = plsc.load_gather(off_v, [iota])
    offs_hi = plsc.load_gather(off_v, [iota + 1])
    sel = (iota == e_mine).astype(jnp.int32)
    off_e = jnp.sum(sel * offs_lo)
    off_next = jnp.sum(sel * offs_hi)
    zstart = off_e + cnt
    zlen = off_next - zstart
    n0 = zlen // 2
    my_start = jnp.where(wid % 2 == 0, zstart, zstart + n0)
    my_len = jnp.where(wid % 2 == 0, n0, zlen - n0)
    my_end = my_start + my_len
    nz = (my_len + ZROWS - 1) // ZROWS

    def zfill(i):
        @pl.when(i < nz)
        def _():
            r = jnp.minimum(my_start + i * ZROWS + iota, my_end - 1)
            pltpu.async_copy(zeros_v, out_hbm.at[plsc.Indices(r)], sem_z)

    def dst_rows(d):
        e = eidx_all[pl.ds(base + d * LANES, LANES)]
        sl = slot_v[pl.ds(d * LANES, LANES)]
        return plsc.load_gather(off_v, [e]) + sl

    out_cp = {}
    for d in range(NDMA):
        b = d % 2
        in_cp[d].wait()
        out_cp[d] = pltpu.async_copy(rows_v.at[b],
                                     out_hbm.at[plsc.Indices(dst_rows(d))],
                                     out_sems[b])
        zfill(jnp.int32(2 * d))
        zfill(jnp.int32(2 * d + 1))
        if d >= 1 and d + 1 < NDMA:
            out_cp[d - 1].wait()
            in_cp[d + 1] = pltpu.async_copy(
                th_hbm.at[pl.ds(base + (d + 1) * K, K)],
                rows_v.at[(d + 1) % 2], in_sems[(d + 1) % 2])
    out_cp[NDMA - 2].wait()
    out_cp[NDMA - 1].wait()

    @pl.loop(0, nz)
    def _zdrain(i):
        pltpu.make_async_copy(zeros_v, out_hbm.at[plsc.Indices(iota)],
                              sem_z).wait()


@jax.jit
def _dispatch(token_hidden, expert_idx, slot_idx, expert_offsets_padded):
    mesh = plsc.VectorSubcoreMesh(core_axis_name="c", subcore_axis_name="s")
    f = pl.kernel(
        _body,
        out_type=jax.ShapeDtypeStruct((OUT_ROWS, HIDDEN), jnp.float32),
        mesh=mesh,
        compiler_params=pltpu.CompilerParams(needs_layout_passes=False),
        scratch_types=[
            pltpu.VMEM((NUM_TOKENS,), jnp.int32),
            pltpu.VMEM((TOK_PER_W,), jnp.int32),
            pltpu.VMEM((32,), jnp.int32),
            pltpu.VMEM((2, K, HIDDEN), jnp.float32),
            pltpu.VMEM((ZROWS, HIDDEN), jnp.float32),
            pltpu.SemaphoreType.DMA,
            pltpu.SemaphoreType.DMA,
            pltpu.SemaphoreType.DMA,
            pltpu.SemaphoreType.DMA,
            pltpu.SemaphoreType.DMA,
        ],
    )
    return f(token_hidden, expert_idx, slot_idx, expert_offsets_padded)


def kernel(token_hidden, expert_idx, slot_idx, expert_offsets):
    off_pad = jnp.concatenate(
        [expert_offsets.astype(jnp.int32),
         jnp.zeros((32 - expert_offsets.shape[0],), jnp.int32)])
    return _dispatch(token_hidden, expert_idx.astype(jnp.int32),
                     slot_idx.astype(jnp.int32), off_pad)


# speedup vs baseline: 1.1036x; 1.1036x over previous
import jax
import jax.numpy as jnp
from jax import lax
from jax.experimental import pallas as pl
from jax.experimental.pallas import tpu as pltpu
from jax.experimental.pallas import tpu_sc as plsc

NUM_TOKENS = 8192
HIDDEN = 2048
NUM_EXPERTS = 16
CAPACITY = 1024
OUT_ROWS = NUM_EXPERTS * CAPACITY

NC = 2
NS = 16
NW = NC * NS
LANES = 16

TOK_PER_W = NUM_TOKENS // NW
K = 16
NDMA = TOK_PER_W // K
ZROWS = 16


def _body(th_hbm, eidx_hbm, slot_hbm, off_hbm, out_hbm,
          eidx_all, slot_v, off_v, idx_v, rows_v, zeros_v,
          sem_in0, sem_in1, sem_out0, sem_out1, sem_z):
    c = lax.axis_index("c")
    s = lax.axis_index("s")
    wid = s * NC + c
    base = wid * TOK_PER_W

    iota = lax.iota(jnp.int32, LANES)
    zero16 = jnp.zeros((LANES,), jnp.float32)
    ones16 = jnp.ones((LANES,), jnp.int32)

    in_sems = (sem_in0, sem_in1)
    out_sems = (sem_out0, sem_out1)

    in_cp = {
        0: pltpu.async_copy(th_hbm.at[pl.ds(base, K)], rows_v.at[0],
                            in_sems[0]),
        1: pltpu.async_copy(th_hbm.at[pl.ds(base + K, K)], rows_v.at[1],
                            in_sems[1]),
    }

    pltpu.sync_copy(eidx_hbm, eidx_all)
    pltpu.sync_copy(slot_hbm.at[pl.ds(base, TOK_PER_W)], slot_v)
    pltpu.sync_copy(off_hbm, off_v)

    for r in range(ZROWS):
        @pl.loop(0, HIDDEN // LANES, unroll=4)
        def _zinit(i, r=r):
            zeros_v[r, pl.ds(i * LANES, LANES)] = zero16

    e_mine = wid // 2

    @pl.loop(0, NUM_TOKENS // LANES, unroll=4,
             init_carry=jnp.zeros((LANES,), jnp.int32))
    def _count(i, cnt_vec):
        e = eidx_all[pl.ds(i * LANES, LANES)]
        return cnt_vec + (e == e_mine).astype(jnp.int32)

    cnt = jnp.sum(_count)

    offs_lo = plsc.load_gather(off_v, [iota])
    offs_hi = plsc.load_gather(off_v, [iota + 1])
    sel = (iota == e_mine).astype(jnp.int32)
    off_e = jnp.sum(sel * offs_lo)
    off_next = jnp.sum(sel * offs_hi)
    zstart = off_e + cnt
    zlen = off_next - zstart
    n0 = zlen // 2
    my_start = jnp.where(wid % 2 == 0, zstart, zstart + n0)
    my_len = jnp.where(wid % 2 == 0, n0, zlen - n0)
    my_end = my_start + my_len
    nz = (my_len + ZROWS - 1) // ZROWS

    @pl.loop(0, nz)
    def _zfill(i):
        r = jnp.minimum(my_start + i * ZROWS + iota, my_end - 1)
        pltpu.async_copy(zeros_v, out_hbm.at[plsc.Indices(r)], sem_z)

    for d in range(NDMA):
        e = eidx_all[pl.ds(base + d * LANES, LANES)]
        sl = slot_v[pl.ds(d * LANES, LANES)]
        idx_v[d] = plsc.load_gather(off_v, [e]) + sl

    out_cp = {}
    for d in range(NDMA):
        b = d % 2
        in_cp[d].wait()
        out_cp[d] = pltpu.async_copy(rows_v.at[b],
                                     out_hbm.at[idx_v.at[d]],
                                     out_sems[b])
        if d >= 1 and d + 1 < NDMA:
            out_cp[d - 1].wait()
            in_cp[d + 1] = pltpu.async_copy(
                th_hbm.at[pl.ds(base + (d + 1) * K, K)],
                rows_v.at[(d + 1) % 2], in_sems[(d + 1) % 2])
    out_cp[NDMA - 2].wait()
    out_cp[NDMA - 1].wait()

    @pl.loop(0, nz)
    def _zdrain(i):
        pltpu.make_async_copy(zeros_v, out_hbm.at[plsc.Indices(iota)],
                              sem_z).wait()


@jax.jit
def _dispatch(token_hidden, expert_idx, slot_idx, expert_offsets_padded):
    mesh = plsc.VectorSubcoreMesh(core_axis_name="c", subcore_axis_name="s")
    f = pl.kernel(
        _body,
        out_type=jax.ShapeDtypeStruct((OUT_ROWS, HIDDEN), jnp.float32),
        mesh=mesh,
        compiler_params=pltpu.CompilerParams(needs_layout_passes=False),
        scratch_types=[
            pltpu.VMEM((NUM_TOKENS,), jnp.int32),
            pltpu.VMEM((TOK_PER_W,), jnp.int32),
            pltpu.VMEM((32,), jnp.int32),
            pltpu.VMEM((NDMA, K), jnp.int32),
            pltpu.VMEM((2, K, HIDDEN), jnp.float32),
            pltpu.VMEM((ZROWS, HIDDEN), jnp.float32),
            pltpu.SemaphoreType.DMA,
            pltpu.SemaphoreType.DMA,
            pltpu.SemaphoreType.DMA,
            pltpu.SemaphoreType.DMA,
            pltpu.SemaphoreType.DMA,
        ],
    )
    return f(token_hidden, expert_idx, slot_idx, expert_offsets_padded)


def kernel(token_hidden, expert_idx, slot_idx, expert_offsets):
    off_pad = jnp.concatenate(
        [expert_offsets.astype(jnp.int32),
         jnp.zeros((32 - expert_offsets.shape[0],), jnp.int32)])
    return _dispatch(token_hidden, expert_idx.astype(jnp.int32),
                     slot_idx.astype(jnp.int32), off_pad)
